# Initial kernel scaffold; baseline (speedup 1.0000x reference)
import jax
import jax.numpy as jnp
from jax.experimental import pallas as pl


def kernel(nxyz, params, nbr_list):
    raise NotImplementedError("write your pallas kernel here")


# speedup vs baseline: 1.6866x; 1.6866x over previous
import jax
import jax.numpy as jnp
from jax import lax
from jax.experimental import pallas as pl
from jax.experimental.pallas import tpu as pltpu
from jax.experimental.pallas import tpu_sc as plsc

N = 10000
E = 320000
F = 128
NG = 20
CUTOFF = 5.0
NMOL = 10

NWORK = 32
EPW = E // NWORK
CH = 80
NCH = EPW // CH
NSUB = 16
NPAD = 10240
RPS = NPAD // NSUB

_mesh = plsc.VectorSubcoreMesh(core_axis_name="c", subcore_axis_name="s",
                               num_cores=2, num_subcores=16)


def _edge_body(src_hbm, dst_hbm, pv_hbm, w_hbm, unit_hbm, zeros_hbm,
               out_hbm,
               idx_s, idx_d, pv_r, w_r, unit_r, msg, acc, sem):
    cid = lax.axis_index("c")
    sid = lax.axis_index("s")
    wid = sid * 2 + cid
    pltpu.sync_copy(zeros_hbm.at[pl.ds(sid * RPS, RPS)],
                    acc.at[pl.ds(sid * RPS, RPS)])
    plsc.subcore_barrier()
    base = wid * EPW

    def chunk(i, carry):
        off = base + i * CH
        pltpu.sync_copy(src_hbm.at[pl.ds(off, CH)], idx_s)
        pltpu.sync_copy(dst_hbm.at[pl.ds(off, CH)], idx_d)
        pltpu.async_copy(pv_hbm.at[idx_d], pv_r, sem).wait()
        pltpu.sync_copy(w_hbm.at[pl.ds(off, CH), :], w_r)
        pltpu.sync_copy(unit_hbm.at[pl.ds(off * 4, CH * 4)], unit_r)

        def edge(e, c2):
            ph = [pv_r[e, pl.ds(16 * t, 16)] for t in range(6)]
            wv = [w_r[e, pl.ds(16 * t, 16)] for t in range(6)]
            vv = [pv_r[e, pl.ds(96 + 16 * t, 16)] for t in range(6)]
            msg[e, pl.ds(0, 16)] = ph[0] * wv[0]
            msg[e, pl.ds(16, 16)] = ph[1] * wv[1]
            avs = (ph[2] * wv[2], ph[3] * wv[3])
            avd = (ph[4] * wv[4], ph[5] * wv[5])
            uv = unit_r[pl.ds(e * 4, 16)]
            for k in range(3):
                uk = jnp.full((16,), uv[k], jnp.float32)
                for h in range(2):
                    msg[e, pl.ds(32 + 32 * k + 16 * h, 16)] = (
                        avs[h] * vv[2 * k + h] + avd[h] * uk)
            return c2

        lax.fori_loop(0, CH, edge, 0)
        pltpu.sync_copy(msg, acc.at[idx_s], add=True)
        return carry

    lax.fori_loop(0, NCH, chunk, 0)
    plsc.subcore_barrier()
    pltpu.sync_copy(acc.at[pl.ds(sid * RPS, RPS)],
                    out_hbm.at[cid, pl.ds(sid * RPS, RPS)])


_edge_call = pl.kernel(
    _edge_body,
    out_type=jax.ShapeDtypeStruct((2, NPAD, 128), jnp.float32),
    mesh=_mesh,
    scratch_types=[
        pltpu.VMEM((CH,), jnp.int32),
        pltpu.VMEM((CH,), jnp.int32),
        pltpu.VMEM((CH, 256), jnp.float32),
        pltpu.VMEM((CH, 96), jnp.float32),
        pltpu.VMEM((CH * 4,), jnp.float32),
        pltpu.VMEM((CH, 128), jnp.float32),
        pltpu.VMEM_SHARED((NPAD, 128), jnp.float32),
        pltpu.SemaphoreType.DMA,
    ],
)


def _swish(x):
    return x * jax.nn.sigmoid(x)


_PERM = []
for _c in range(4):
    for _t in range(3):
        _PERM += list(range(128 * _t + 32 * _c, 128 * _t + 32 * _c + 32))


def kernel(nxyz, params, nbr_list):
    z = nxyz[:, 0].astype(jnp.int32)
    xyz = nxyz[:, 1:]
    src = nbr_list[:, 0].astype(jnp.int32)
    dst = nbr_list[:, 1].astype(jnp.int32)
    r_ij = xyz[dst] - xyz[src]
    d = jnp.sqrt(jnp.sum(r_ij ** 2, axis=-1) + 1e-12)
    unit = r_ij / d[:, None]
    unit4 = jnp.pad(unit, ((0, 0), (0, 1))).reshape(-1)
    offsets = jnp.linspace(0.0, CUTOFF, NG)
    coeff = -0.5 / (offsets[1] - offsets[0]) ** 2
    e_rbf = jnp.exp(coeff * (d[:, None] - offsets[None, :]) ** 2)
    f_cut = 0.5 * (jnp.cos(jnp.pi * d / CUTOFF) + 1.0) * (d < CUTOFF)
    s = params["embed"][z]
    v = jnp.zeros((N, 3, F), jnp.float32)
    zeros_acc = jnp.zeros((NPAD, 128), jnp.float32)
    pad64 = jnp.zeros((N, 64), jnp.float32)
    perm = jnp.asarray(_PERM)

    for lp in params["layers"]:
        phi = _swish(s @ lp["phi_W1"] + lp["phi_b1"]) @ lp["phi_W2"][:, perm] \
            + lp["phi_b2"][perm]
        w_all = (e_rbf @ lp["dist_W"][:, perm] + lp["dist_b"][perm]) \
            * f_cut[:, None]
        ds_parts = []
        dv_parts = []
        for c in range(4):
            phi_c = phi[:, 96 * c:96 * c + 96]
            w_c = w_all[:, 96 * c:96 * c + 96]
            v_c = v[:, :, 32 * c:32 * c + 32].reshape(N, 96)
            pv = jnp.concatenate([phi_c, v_c, pad64], axis=1)
            part = _edge_call(src, dst, pv, w_c, unit4, zeros_acc)
            tot = part[0, :N] + part[1, :N]
            ds_parts.append(tot[:, :32])
            dv_parts.append(tot[:, 32:].reshape(N, 3, 32))
        s = s + jnp.concatenate(ds_parts, axis=1)
        v = v + jnp.concatenate(dv_parts, axis=2)
        u_v = v @ lp["U"]
        v_v = v @ lp["V"]
        v_norm = jnp.sqrt(jnp.sum(v_v ** 2, axis=1) + 1e-12)
        a = _swish(jnp.concatenate([s, v_norm], axis=-1) @ lp["upd_W1"]
                   + lp["upd_b1"]) @ lp["upd_W2"] + lp["upd_b2"]
        a_vv, a_sv, a_ss = jnp.split(a, 3, axis=-1)
        s = s + a_sv * jnp.sum(u_v * v_v, axis=1) + a_ss
        v = v + a_vv[:, None, :] * u_v

    atom_e = _swish(s @ params["ro_W1"] + params["ro_b1"]) @ params["ro_W2"] \
        + params["ro_b2"]
    energy = jnp.sum(atom_e.reshape(NMOL, N // NMOL, 1), axis=1)
    return energy, s
